# Initial kernel scaffold; baseline (speedup 1.0000x reference)
import jax
import jax.numpy as jnp
from jax.experimental import pallas as pl


def kernel(z, edge_index):
    raise NotImplementedError("write your pallas kernel here")


# speedup vs baseline: 4.7968x; 4.7968x over previous
import functools

import jax
import jax.numpy as jnp
from jax import lax
from jax.experimental import pallas as pl
from jax.experimental.pallas import tpu as pltpu
from jax.experimental.pallas import tpu_sc as plsc

N_NODES = 10000
D = 128
E = 320000
NC, NS = 2, 16
NW = NC * NS
EPW = E // NW
C = 400
NCHUNK = EPW // C
G = 16
LANES = 16


def _sc_body(z_hbm, src_hbm, dst_hbm, out_hbm,
             idx_s, idx_d, rows_s, rows_d, res, tr, sem_s, sem_d):
    wid = lax.axis_index("c") * NS + lax.axis_index("s")
    base = wid * EPW

    def chunk(g, carry):
        off = base + g * C
        pltpu.sync_copy(src_hbm.at[pl.ds(off, C)], idx_s)
        pltpu.sync_copy(dst_hbm.at[pl.ds(off, C)], idx_d)
        cp1 = pltpu.async_copy(z_hbm.at[idx_s], rows_s, sem_s)
        cp2 = pltpu.async_copy(z_hbm.at[idx_d], rows_d, sem_d)
        cp1.wait()
        cp2.wait()

        lane16 = lax.iota(jnp.int32, LANES) * LANES

        def grp(t, carry2):
            e0 = t * G
            for j in range(G):
                acc = (rows_s[e0 + j, pl.ds(0, LANES)]
                       * rows_d[e0 + j, pl.ds(0, LANES)])
                for k in range(1, D // LANES):
                    acc = acc + (rows_s[e0 + j, pl.ds(k * LANES, LANES)]
                                 * rows_d[e0 + j, pl.ds(k * LANES, LANES)])
                plsc.store_scatter(tr, [lane16 + j], acc)
            dots = tr[pl.ds(0, LANES)]
            for c in range(1, G):
                dots = dots + tr[pl.ds(c * LANES, LANES)]
            res[pl.ds(e0, LANES)] = 1.0 / (1.0 + jnp.exp(-dots))
            return carry2

        lax.fori_loop(0, C // G, grp, 0)
        pltpu.sync_copy(res, out_hbm.at[pl.ds(off, C)])
        return carry

    lax.fori_loop(0, NCHUNK, chunk, 0)


_mesh = plsc.VectorSubcoreMesh(
    core_axis_name="c", subcore_axis_name="s", num_cores=NC, num_subcores=NS)

_ip_kernel = functools.partial(
    pl.kernel,
    out_type=jax.ShapeDtypeStruct((E,), jnp.float32),
    mesh=_mesh,
    compiler_params=pltpu.CompilerParams(needs_layout_passes=False),
    scratch_types=[
        pltpu.VMEM((C,), jnp.int32),
        pltpu.VMEM((C,), jnp.int32),
        pltpu.VMEM((C, D), jnp.float32),
        pltpu.VMEM((C, D), jnp.float32),
        pltpu.VMEM((C,), jnp.float32),
        pltpu.VMEM((G * LANES,), jnp.float32),
        pltpu.SemaphoreType.DMA,
        pltpu.SemaphoreType.DMA,
    ],
)(_sc_body)


def kernel(z, edge_index):
    src = edge_index[0]
    dst = edge_index[1]
    return _ip_kernel(z, src, dst)


# speedup vs baseline: 7.8143x; 1.6291x over previous
import functools

import jax
import jax.numpy as jnp
from jax import lax
from jax.experimental import pallas as pl
from jax.experimental.pallas import tpu as pltpu
from jax.experimental.pallas import tpu_sc as plsc

N_NODES = 10000
D = 128
E = 320000
NC, NS = 2, 16
NW = NC * NS
EPW = E // NW
C = 80
NCHUNK = EPW // C
NBUF = 5
G = 16
LANES = 16


def _sc_body(z_hbm, src_hbm, dst_hbm, out_hbm,
             idx_s, idx_d, rows_s, rows_d, outb, tr,
             gsem_s, gsem_d, osem):
    wid = lax.axis_index("c") * NS + lax.axis_index("s")
    base = wid * EPW

    pltpu.sync_copy(src_hbm.at[pl.ds(base, EPW)], idx_s)
    pltpu.sync_copy(dst_hbm.at[pl.ds(base, EPW)], idx_d)

    lane16 = lax.iota(jnp.int32, LANES) * LANES

    def gather_pair(g, b):
        return (
            pltpu.make_async_copy(
                z_hbm.at[idx_s.at[pl.ds(g * C, C)]], rows_s.at[b],
                gsem_s.at[b]),
            pltpu.make_async_copy(
                z_hbm.at[idx_d.at[pl.ds(g * C, C)]], rows_d.at[b],
                gsem_d.at[b]),
        )

    def out_copy(g, b):
        return pltpu.make_async_copy(
            outb.at[b], out_hbm.at[pl.ds(base + g * C, C)], osem.at[b])

    def compute_chunk(b):
        rs = rows_s.at[b]
        rd = rows_d.at[b]

        def grp(t, carry):
            e0 = t * G
            for j in range(G):
                acc = rs[e0 + j, pl.ds(0, LANES)] * rd[e0 + j, pl.ds(0, LANES)]
                for k in range(1, D // LANES):
                    acc = acc + (rs[e0 + j, pl.ds(k * LANES, LANES)]
                                 * rd[e0 + j, pl.ds(k * LANES, LANES)])
                plsc.store_scatter(tr, [lane16 + j], acc)
            dots = tr[pl.ds(0, LANES)]
            for c in range(1, G):
                dots = dots + tr[pl.ds(c * LANES, LANES)]
            outb[b, pl.ds(e0, LANES)] = 1.0 / (1.0 + jnp.exp(-dots))
            return carry

        lax.fori_loop(0, C // G, grp, 0)

    for b in range(NBUF):
        for cp in gather_pair(b, b):
            cp.start()

    def outer(gg, carry):
        for b in range(NBUF):
            g = gg * NBUF + b
            for cp in gather_pair(g, b):
                cp.wait()

            @pl.when(g >= NBUF)
            def _():
                out_copy(g - NBUF, b).wait()

            compute_chunk(b)
            out_copy(g, b).start()

            @pl.when(g + NBUF < NCHUNK)
            def _():
                for cp in gather_pair(g + NBUF, b):
                    cp.start()
        return carry

    lax.fori_loop(0, NCHUNK // NBUF, outer, 0)

    for b in range(NBUF):
        out_copy(NCHUNK - NBUF + b, b).wait()


_mesh = plsc.VectorSubcoreMesh(
    core_axis_name="c", subcore_axis_name="s", num_cores=NC, num_subcores=NS)

_ip_kernel = functools.partial(
    pl.kernel,
    out_type=jax.ShapeDtypeStruct((E,), jnp.float32),
    mesh=_mesh,
    compiler_params=pltpu.CompilerParams(needs_layout_passes=False),
    scratch_types=[
        pltpu.VMEM((EPW,), jnp.int32),
        pltpu.VMEM((EPW,), jnp.int32),
        pltpu.VMEM((NBUF, C, D), jnp.float32),
        pltpu.VMEM((NBUF, C, D), jnp.float32),
        pltpu.VMEM((NBUF, C), jnp.float32),
        pltpu.VMEM((G * LANES,), jnp.float32),
        pltpu.SemaphoreType.DMA((NBUF,)),
        pltpu.SemaphoreType.DMA((NBUF,)),
        pltpu.SemaphoreType.DMA((NBUF,)),
    ],
)(_sc_body)


def kernel(z, edge_index):
    src = edge_index[0]
    dst = edge_index[1]
    return _ip_kernel(z, src, dst)
